# baseline (speedup 1.0000x reference)
import functools
import math

import jax
import jax.numpy as jnp
from jax import lax
from jax.experimental import pallas as pl
from jax.experimental.pallas import tpu as pltpu
from jax.experimental.pallas import tpu_sc as plsc

_VOCAB = 1000
_SEQ = 200
_BATCH = 16384
_NW = 32
_RPT = _BATCH // _NW
_GROUPS = _RPT // 16
_NCHUNK = 8
_GPC = _GROUPS // _NCHUNK
_CHUNK_W = _GPC * 16 * _SEQ
_TILE_W = _RPT * _SEQ
_LOG_SEQ = math.log(float(_SEQ))
_U = 8


def _entropy_sc(x_hbm, tab_hbm, out_hbm, tokens_v, tab_v, hist_v, out_v, sems):
    wid = lax.axis_index("s") * 2 + lax.axis_index("c")
    base_w = wid * _TILE_W

    copies = []
    for ci in range(_NCHUNK):
        copies.append(
            pltpu.async_copy(
                x_hbm.at[pl.ds(base_w + ci * _CHUNK_W, _CHUNK_W)],
                tokens_v.at[pl.ds(ci * _CHUNK_W, _CHUNK_W)],
                sems.at[ci],
            )
        )
    pltpu.sync_copy(tab_hbm, tab_v)

    lane = lax.iota(jnp.int32, 16)
    zeros_i = jnp.zeros((16,), jnp.int32)
    ones_i = jnp.ones((16,), jnp.int32)

    def zero_hist(k, carry):
        hist_v[pl.ds(k * 16, 16)] = zeros_i
        return carry

    lax.fori_loop(0, (16 * _VOCAB) // 16, zero_hist, 0)

    def per_group(g, carry):
        gbase = g * (16 * _SEQ)

        def count_pass(i, c2):
            for k in range(_U):
                tok = tokens_v[pl.ds(gbase + (i * _U + k) * 16, 16)]
                plsc.addupdate_scatter(hist_v, [tok * 16 + lane], ones_i)
            return c2

        lax.fori_loop(0, _SEQ // _U, count_pass, 0)

        def reduce_pass(i, accs):
            accs = list(accs)
            for k in range(_U):
                tok = tokens_v[pl.ds(gbase + (i * _U + k) * 16, 16)]
                cnt = plsc.load_gather(hist_v, [tok * 16 + lane])
                accs[k % 4] = accs[k % 4] + plsc.load_gather(
                    tab_v, [cnt * 16 + lane]
                )
            return tuple(accs)

        zf = jnp.zeros((16,), jnp.float32)
        a0, a1, a2, a3 = lax.fori_loop(
            0, _SEQ // _U, reduce_pass, (zf, zf, zf, zf)
        )
        acc = (a0 + a1) + (a2 + a3)

        def clear_pass(i, c2):
            for k in range(_U):
                tok = tokens_v[pl.ds(gbase + (i * _U + k) * 16, 16)]
                plsc.store_scatter(hist_v, [tok * 16 + lane], zeros_i)
            return c2

        lax.fori_loop(0, _SEQ // _U, clear_pass, 0)

        out_v[pl.ds(g * 16, 16)] = _LOG_SEQ - acc * (1.0 / _SEQ)
        return carry

    for ci in range(_NCHUNK):
        copies[ci].wait()
        lax.fori_loop(0, _GPC, lambda gg, c, ci=ci: per_group(ci * _GPC + gg, c), 0)

    pltpu.sync_copy(out_v, out_hbm.at[pl.ds(wid * _RPT, _RPT)])


def kernel(x):
    xg = x.reshape(_BATCH // 16, 16, _SEQ).swapaxes(1, 2).reshape(-1)
    c = jnp.arange(256, dtype=jnp.float32)
    tab = jnp.repeat(jnp.log(jnp.maximum(c, 1.0)), 16)
    mesh = plsc.VectorSubcoreMesh(core_axis_name="c", subcore_axis_name="s")
    run = functools.partial(
        pl.kernel,
        mesh=mesh,
        out_type=jax.ShapeDtypeStruct((_BATCH,), jnp.float32),
        scratch_types=[
            pltpu.VMEM((_TILE_W,), jnp.int32),
            pltpu.VMEM((256 * 16,), jnp.float32),
            pltpu.VMEM((16 * _VOCAB,), jnp.int32),
            pltpu.VMEM((_RPT,), jnp.float32),
            pltpu.SemaphoreType.DMA((_NCHUNK,)),
        ],
        compiler_params=pltpu.CompilerParams(needs_layout_passes=False),
    )(_entropy_sc)
    return run(xg, tab)[:, None]


# speedup vs baseline: 2.4816x; 2.4816x over previous
import functools
import math

import jax
import jax.numpy as jnp
from jax import lax
from jax.experimental import pallas as pl
from jax.experimental.pallas import tpu as pltpu
from jax.experimental.pallas import tpu_sc as plsc

_VOCAB = 1000
_SEQ = 200
_BATCH = 16384
_NW = 32
_RPT = _BATCH // _NW
_NCHUNK = 8
_RPC = _RPT // _NCHUNK
_CHUNK_W = _RPC * _SEQ
_TILE_W = _RPT * _SEQ
_NV = _SEQ // 16 + 1
_LOG_SEQ = math.log(float(_SEQ))


def _entropy_sc(x_hbm, tab_hbm, out_hbm, tokens_v, tab_v, hist_v, out_v, sems):
    wid = lax.axis_index("s") * 2 + lax.axis_index("c")
    base_w = wid * _TILE_W

    copies = []
    for ci in range(_NCHUNK):
        copies.append(
            pltpu.async_copy(
                x_hbm.at[pl.ds(base_w + ci * _CHUNK_W, _CHUNK_W)],
                tokens_v.at[pl.ds(ci * _CHUNK_W, _CHUNK_W)],
                sems.at[ci],
            )
        )
    pltpu.sync_copy(tab_hbm, tab_v)

    lane = lax.iota(jnp.int32, 16)
    zeros_i = jnp.zeros((16,), jnp.int32)
    ones_i = jnp.ones((16,), jnp.int32)
    dummy = _VOCAB + lane
    tail_sel = lane < 8
    lane0 = lane == 0

    def zero_hist(k, carry):
        hist_v[pl.ds(k * 16, 16)] = zeros_i
        return carry

    lax.fori_loop(0, _HIST_W // 16, zero_hist, 0)

    def per_row(r, carry):
        rbase = r * _SEQ
        toks = [tokens_v[pl.ds(rbase + 16 * k, 16)] for k in range(_NV - 1)]
        toks.append(
            jnp.where(tail_sel, tokens_v[pl.ds(rbase + _SEQ - 8, 16)], dummy)
        )

        for t in toks:
            plsc.addupdate_scatter(hist_v, [t], ones_i)

        zf = jnp.zeros((16,), jnp.float32)
        accs = [zf, zf, zf, zf]
        for k, t in enumerate(toks):
            cnt = plsc.load_gather(hist_v, [t])
            accs[k % 4] = accs[k % 4] + plsc.load_gather(
                tab_v, [cnt * 16 + lane]
            )

        for t in toks:
            plsc.store_scatter(hist_v, [t], zeros_i)

        s = jnp.sum((accs[0] + accs[1]) + (accs[2] + accs[3]))
        h = _LOG_SEQ - s * (1.0 / _SEQ)
        plsc.store_scatter(out_v, [zeros_i + r], jnp.zeros((16,), jnp.float32) + h,
                           mask=lane0)
        return carry

    for ci in range(_NCHUNK):
        copies[ci].wait()
        lax.fori_loop(0, _RPC, lambda rr, c, ci=ci: per_row(ci * _RPC + rr, c), 0)

    pltpu.sync_copy(out_v, out_hbm.at[pl.ds(wid * _RPT, _RPT)])


_HIST_W = 1024


def kernel(x):
    c = jnp.arange(256, dtype=jnp.float32)
    tab = jnp.repeat(jnp.log(jnp.maximum(c, 1.0)), 16)
    mesh = plsc.VectorSubcoreMesh(core_axis_name="c", subcore_axis_name="s")
    run = functools.partial(
        pl.kernel,
        mesh=mesh,
        out_type=jax.ShapeDtypeStruct((_BATCH,), jnp.float32),
        scratch_types=[
            pltpu.VMEM((_TILE_W + 16,), jnp.int32),
            pltpu.VMEM((256 * 16,), jnp.float32),
            pltpu.VMEM((_HIST_W,), jnp.int32),
            pltpu.VMEM((_RPT,), jnp.float32),
            pltpu.SemaphoreType.DMA((_NCHUNK,)),
        ],
        compiler_params=pltpu.CompilerParams(needs_layout_passes=False),
    )(_entropy_sc)
    return run(x.reshape(-1), tab)[:, None]


# speedup vs baseline: 2.6634x; 1.0733x over previous
import functools
import math

import jax
import jax.numpy as jnp
from jax import lax
from jax.experimental import pallas as pl
from jax.experimental.pallas import tpu as pltpu
from jax.experimental.pallas import tpu_sc as plsc

_VOCAB = 1000
_SEQ = 200
_BATCH = 16384
_NW = 32
_RPT = _BATCH // _NW
_NCHUNK = 8
_RPC = _RPT // _NCHUNK
_CHUNK_W = _RPC * _SEQ
_TILE_W = _RPT * _SEQ
_NV = _SEQ // 16 + 1
_LOG_SEQ = math.log(float(_SEQ))


def _entropy_sc(x_hbm, tab_hbm, out_hbm, tokens_v, tab_v, hist_v, hist2_v,
                out_v, sems):
    wid = lax.axis_index("s") * 2 + lax.axis_index("c")
    base_w = wid * _TILE_W

    copies = []
    for ci in range(_NCHUNK):
        copies.append(
            pltpu.async_copy(
                x_hbm.at[pl.ds(base_w + ci * _CHUNK_W, _CHUNK_W)],
                tokens_v.at[pl.ds(ci * _CHUNK_W, _CHUNK_W)],
                sems.at[ci],
            )
        )
    pltpu.sync_copy(tab_hbm, tab_v)

    lane = lax.iota(jnp.int32, 16)
    zeros_i = jnp.zeros((16,), jnp.int32)
    ones_i = jnp.ones((16,), jnp.int32)
    dummy = _VOCAB + lane
    tail_sel = lane < 8
    lane0 = lane == 0

    def zero_hist(k, carry):
        hist_v[pl.ds(k * 16, 16)] = zeros_i
        hist2_v[pl.ds(k * 16, 16)] = zeros_i
        return carry

    lax.fori_loop(0, _HIST_W // 16, zero_hist, 0)

    def load_row(r):
        rbase = r * _SEQ
        toks = [tokens_v[pl.ds(rbase + 16 * k, 16)] for k in range(_NV - 1)]
        toks.append(
            jnp.where(tail_sel, tokens_v[pl.ds(rbase + _SEQ - 8, 16)], dummy)
        )
        return toks

    def per_row_pair(p, carry):
        r = p * 2
        toks_a = load_row(r)
        toks_b = load_row(r + 1)

        for ta, tb in zip(toks_a, toks_b):
            plsc.addupdate_scatter(hist_v, [ta], ones_i)
            plsc.addupdate_scatter(hist2_v, [tb], ones_i)

        zf = jnp.zeros((16,), jnp.float32)
        accs_a = [zf, zf, zf, zf]
        accs_b = [zf, zf, zf, zf]
        for k, (ta, tb) in enumerate(zip(toks_a, toks_b)):
            cnt_a = plsc.load_gather(hist_v, [ta])
            cnt_b = plsc.load_gather(hist2_v, [tb])
            accs_a[k % 4] = accs_a[k % 4] + plsc.load_gather(
                tab_v, [cnt_a * 16 + lane]
            )
            accs_b[k % 4] = accs_b[k % 4] + plsc.load_gather(
                tab_v, [cnt_b * 16 + lane]
            )

        for ta, tb in zip(toks_a, toks_b):
            plsc.store_scatter(hist_v, [ta], zeros_i)
            plsc.store_scatter(hist2_v, [tb], zeros_i)

        s_a = jnp.sum((accs_a[0] + accs_a[1]) + (accs_a[2] + accs_a[3]))
        s_b = jnp.sum((accs_b[0] + accs_b[1]) + (accs_b[2] + accs_b[3]))
        h_a = _LOG_SEQ - s_a * (1.0 / _SEQ)
        h_b = _LOG_SEQ - s_b * (1.0 / _SEQ)
        plsc.store_scatter(out_v, [zeros_i + r],
                           jnp.zeros((16,), jnp.float32) + h_a, mask=lane0)
        plsc.store_scatter(out_v, [zeros_i + (r + 1)],
                           jnp.zeros((16,), jnp.float32) + h_b, mask=lane0)
        return carry

    for ci in range(_NCHUNK):
        copies[ci].wait()
        lax.fori_loop(
            0, _RPC // 2,
            lambda pp, c, ci=ci: per_row_pair(ci * (_RPC // 2) + pp, c), 0,
        )

    pltpu.sync_copy(out_v, out_hbm.at[pl.ds(wid * _RPT, _RPT)])


_HIST_W = 1024


def kernel(x):
    c = jnp.arange(256, dtype=jnp.float32)
    tab = jnp.repeat(jnp.log(jnp.maximum(c, 1.0)), 16)
    mesh = plsc.VectorSubcoreMesh(core_axis_name="c", subcore_axis_name="s")
    run = functools.partial(
        pl.kernel,
        mesh=mesh,
        out_type=jax.ShapeDtypeStruct((_BATCH,), jnp.float32),
        scratch_types=[
            pltpu.VMEM((_TILE_W + 16,), jnp.int32),
            pltpu.VMEM((256 * 16,), jnp.float32),
            pltpu.VMEM((_HIST_W,), jnp.int32),
            pltpu.VMEM((_HIST_W,), jnp.int32),
            pltpu.VMEM((_RPT,), jnp.float32),
            pltpu.SemaphoreType.DMA((_NCHUNK,)),
        ],
        compiler_params=pltpu.CompilerParams(needs_layout_passes=False),
    )(_entropy_sc)
    return run(x.reshape(-1), tab)[:, None]
